# baseline (speedup 1.0000x reference)
import functools

import jax
import jax.numpy as jnp
import numpy as np
from jax.experimental import pallas as pl

_ROT = ((13, 15, 26, 6), (17, 29, 16, 24))
_TINY = np.float32(np.finfo(np.float32).tiny)


def _rotl(x, r):
    return (x << jnp.uint32(r)) | (x >> jnp.uint32(32 - r))


def _tf_bits(lo):
    ks = (jnp.uint32(0), jnp.uint32(42), jnp.uint32(0x1BD11BDA ^ 42))
    x1 = lo + ks[1]
    x0 = None
    for rnd in range(5):
        for r in _ROT[rnd % 2]:
            x0 = x1 if x0 is None else x0 + x1
            x1 = _rotl(x1, r)
            x1 = x1 ^ x0
        x0 = x0 + ks[(rnd + 1) % 3]
        x1 = x1 + ks[(rnd + 2) % 3] + jnp.uint32(rnd + 1)
    return x0 ^ x1


def _log_unif(bits):
    fb = (bits >> jnp.uint32(9)) | jnp.uint32(0x3F800000)
    u = jax.lax.bitcast_convert_type(fb, jnp.float32) - jnp.float32(1.0)
    u = jnp.maximum(u, _TINY)
    return jnp.log(u)


def _body(p_ref, o_ref, *, n, block_elems):
    pid = pl.program_id(0)
    br = block_elems // 128
    lanes = 128
    p = p_ref[...].reshape(br, lanes)
    row = jax.lax.broadcasted_iota(jnp.uint32, (br, lanes), 0)
    col = jax.lax.broadcasted_iota(jnp.uint32, (br, lanes), 1)
    base = (pid * block_elems).astype(jnp.uint32)
    j = base + row * jnp.uint32(lanes) + col
    t0 = _log_unif(_tf_bits(j))
    t1 = _log_unif(_tf_bits(j + jnp.uint32(n)))
    p0 = (jnp.float32(1.0) - p) + jnp.float32(1e-15)
    p1 = p + jnp.float32(1e-15)
    out = jnp.where(p0 * t1 <= p1 * t0, jnp.float32(1.0), jnp.float32(0.0))
    o_ref[...] = out.reshape(block_elems)


@jax.jit
def kernel(not_infected_probs):
    p = not_infected_probs
    n = p.shape[0]
    be = 512 * 128
    nb = (n + be - 1) // be
    return pl.pallas_call(
        functools.partial(_body, n=n, block_elems=be),
        grid=(nb,),
        in_specs=[pl.BlockSpec((be,), lambda i: (i,))],
        out_specs=pl.BlockSpec((be,), lambda i: (i,)),
        out_shape=jax.ShapeDtypeStruct((n,), jnp.float32),
    )(p)


# speedup vs baseline: 1.0063x; 1.0063x over previous
import functools

import jax
import jax.numpy as jnp
import numpy as np
from jax.experimental import pallas as pl

_ROT = ((13, 15, 26, 6), (17, 29, 16, 24))
_TINY = np.float32(np.finfo(np.float32).tiny)


def _rotl(x, r):
    return (x << jnp.uint32(r)) | (x >> jnp.uint32(32 - r))


def _tf_bits(lo):
    ks = (jnp.uint32(0), jnp.uint32(42), jnp.uint32(0x1BD11BDA ^ 42))
    x1 = lo + ks[1]
    x0 = None
    for rnd in range(5):
        for r in _ROT[rnd % 2]:
            x0 = x1 if x0 is None else x0 + x1
            x1 = _rotl(x1, r)
            x1 = x1 ^ x0
        x0 = x0 + ks[(rnd + 1) % 3]
        x1 = x1 + ks[(rnd + 2) % 3] + jnp.uint32(rnd + 1)
    return x0 ^ x1


def _log_unif(bits):
    fb = (bits >> jnp.uint32(9)) | jnp.uint32(0x3F800000)
    u = jax.lax.bitcast_convert_type(fb, jnp.float32) - jnp.float32(1.0)
    u = jnp.maximum(u, _TINY)
    return jnp.log(u)


def _body(p_ref, o_ref, *, n, block_elems):
    pid = pl.program_id(0)
    br = block_elems // 128
    lanes = 128
    p = p_ref[...].reshape(br, lanes)
    row = jax.lax.broadcasted_iota(jnp.uint32, (br, lanes), 0)
    col = jax.lax.broadcasted_iota(jnp.uint32, (br, lanes), 1)
    base = (pid * block_elems).astype(jnp.uint32)
    j = base + row * jnp.uint32(lanes) + col
    t0 = _log_unif(_tf_bits(j))
    t1 = _log_unif(_tf_bits(j + jnp.uint32(n)))
    p0 = (jnp.float32(1.0) - p) + jnp.float32(1e-15)
    p1 = p + jnp.float32(1e-15)
    out = jnp.where(p0 * t1 <= p1 * t0, jnp.float32(1.0), jnp.float32(0.0))
    o_ref[...] = out.reshape(block_elems)


@jax.jit
def kernel(not_infected_probs):
    p = not_infected_probs
    n = p.shape[0]
    be = 1024 * 128
    nb = (n + be - 1) // be
    return pl.pallas_call(
        functools.partial(_body, n=n, block_elems=be),
        grid=(nb,),
        in_specs=[pl.BlockSpec((be,), lambda i: (i,))],
        out_specs=pl.BlockSpec((be,), lambda i: (i,)),
        out_shape=jax.ShapeDtypeStruct((n,), jnp.float32),
    )(p)


# speedup vs baseline: 1.0170x; 1.0106x over previous
import functools

import jax
import jax.numpy as jnp
import numpy as np
from jax.experimental import pallas as pl

_ROT = ((13, 15, 26, 6), (17, 29, 16, 24))
_TINY = np.float32(np.finfo(np.float32).tiny)


def _rotl(x, r):
    return (x << jnp.uint32(r)) | (x >> jnp.uint32(32 - r))


def _tf_bits(x1):
    ks = (jnp.uint32(0), jnp.uint32(42), jnp.uint32(0x1BD11BDA ^ 42))
    x0 = None
    for rnd in range(5):
        for r in _ROT[rnd % 2]:
            x0 = x1 if x0 is None else x0 + x1
            x1 = _rotl(x1, r)
            x1 = x1 ^ x0
        x0 = x0 + ks[(rnd + 1) % 3]
        x1 = x1 + ks[(rnd + 2) % 3] + jnp.uint32(rnd + 1)
    return x0 ^ x1


def _log_unif(bits):
    fb = (bits >> jnp.uint32(9)) | jnp.uint32(0x3F800000)
    u = jax.lax.bitcast_convert_type(fb, jnp.float32) - jnp.float32(1.0)
    u = jnp.maximum(u, _TINY)
    return jnp.log(u)


def _body(p_ref, o_ref, *, n, block_elems):
    pid = pl.program_id(0)
    br = block_elems // 128
    lanes = 128
    p = p_ref[...].reshape(br, lanes)
    row = jax.lax.broadcasted_iota(jnp.uint32, (br, lanes), 0)
    col = jax.lax.broadcasted_iota(jnp.uint32, (br, lanes), 1)
    base42 = (pid * block_elems + 42).astype(jnp.uint32)
    x1a = base42 + (row * jnp.uint32(lanes) + col)
    t0 = _log_unif(_tf_bits(x1a))
    t1 = _log_unif(_tf_bits(x1a + jnp.uint32(n)))
    p0 = (jnp.float32(1.0) - p) + jnp.float32(1e-15)
    p1 = p + jnp.float32(1e-15)
    out = jnp.where(p0 * t1 <= p1 * t0, jnp.float32(1.0), jnp.float32(0.0))
    o_ref[...] = out.reshape(block_elems)


@jax.jit
def kernel(not_infected_probs):
    p = not_infected_probs
    n = p.shape[0]
    be = 1024 * 128
    nb = (n + be - 1) // be
    return pl.pallas_call(
        functools.partial(_body, n=n, block_elems=be),
        grid=(nb,),
        in_specs=[pl.BlockSpec((be,), lambda i: (i,))],
        out_specs=pl.BlockSpec((be,), lambda i: (i,)),
        out_shape=jax.ShapeDtypeStruct((n,), jnp.float32),
    )(p)


# speedup vs baseline: 1.0192x; 1.0022x over previous
import functools

import jax
import jax.numpy as jnp
import numpy as np
from jax.experimental import pallas as pl
from jax.experimental.pallas import tpu as pltpu

_ROT = ((13, 15, 26, 6), (17, 29, 16, 24))
_TINY = np.float32(np.finfo(np.float32).tiny)


def _rotl(x, r):
    return (x << jnp.uint32(r)) | (x >> jnp.uint32(32 - r))


def _tf_bits(x1):
    ks = (jnp.uint32(0), jnp.uint32(42), jnp.uint32(0x1BD11BDA ^ 42))
    x0 = None
    for rnd in range(5):
        for r in _ROT[rnd % 2]:
            x0 = x1 if x0 is None else x0 + x1
            x1 = _rotl(x1, r)
            x1 = x1 ^ x0
        x0 = x0 + ks[(rnd + 1) % 3]
        x1 = x1 + ks[(rnd + 2) % 3] + jnp.uint32(rnd + 1)
    return x0 ^ x1


def _log_unif(bits):
    fb = (bits >> jnp.uint32(9)) | jnp.uint32(0x3F800000)
    u = jax.lax.bitcast_convert_type(fb, jnp.float32) - jnp.float32(1.0)
    u = jnp.maximum(u, _TINY)
    return jnp.log(u)


def _body(p_ref, o_ref, rc_ref, *, n, block_elems):
    pid = pl.program_id(0)
    br = block_elems // 128
    lanes = 128
    p = p_ref[...].reshape(br, lanes)

    @pl.when(pid == 0)
    def _():
        row = jax.lax.broadcasted_iota(jnp.uint32, (br, lanes), 0)
        col = jax.lax.broadcasted_iota(jnp.uint32, (br, lanes), 1)
        rc_ref[...] = row * jnp.uint32(lanes) + col

    base42 = (pid * block_elems + 42).astype(jnp.uint32)
    x1a = base42 + rc_ref[...]
    t0 = _log_unif(_tf_bits(x1a))
    t1 = _log_unif(_tf_bits(x1a + jnp.uint32(n)))
    p0 = (jnp.float32(1.0) - p) + jnp.float32(1e-15)
    p1 = p + jnp.float32(1e-15)
    out = jnp.where(p0 * t1 <= p1 * t0, jnp.float32(1.0), jnp.float32(0.0))
    o_ref[...] = out.reshape(block_elems)


@jax.jit
def kernel(not_infected_probs):
    p = not_infected_probs
    n = p.shape[0]
    be = 1024 * 128
    nb = (n + be - 1) // be
    return pl.pallas_call(
        functools.partial(_body, n=n, block_elems=be),
        grid=(nb,),
        in_specs=[pl.BlockSpec((be,), lambda i: (i,))],
        out_specs=pl.BlockSpec((be,), lambda i: (i,)),
        out_shape=jax.ShapeDtypeStruct((n,), jnp.float32),
        scratch_shapes=[pltpu.VMEM((be // 128, 128), jnp.uint32)],
    )(p)
